# baseline (speedup 1.0000x reference)
import functools

import jax
import jax.numpy as jnp
from jax import lax
from jax.experimental import pallas as pl
from jax.experimental.pallas import tpu as pltpu
from jax.experimental.pallas import tpu_sc as plsc

N = 10000
D = 128
H = 64
C = 40
K = 10
ALPHA = 0.1
E = 320000
CP = 48

NSUB = 16
NCORE = 2
NP = 10112
RPS = NP // NSUB
QR = RPS // 8
NW = NCORE * NSUB
CHUNK = 128
NCH = 80
EP = NW * NCH * CHUNK
DUMMY = N
NBUF = 4
NG = NCH // NBUF


def _mlp_body(x_ref, w1_ref, b1_ref, w2_ref, b2_ref, o_ref):
    h = jnp.dot(x_ref[...], w1_ref[...], preferred_element_type=jnp.float32)
    h = jnp.maximum(h + b1_ref[...], 0.0)
    o_ref[...] = jnp.dot(h, w2_ref[...], preferred_element_type=jnp.float32) + b2_ref[...]


def _prep_body(a0_ref, a1_ref, h_ref, g_ref, z_ref, u_ref, dinv_ref):
    deg = a0_ref[:, 0:1] + a1_ref[:, 0:1] + 1.0
    dinv = lax.rsqrt(deg)
    u_ref[...] = jnp.broadcast_to((1.0 - ALPHA) / deg, (NP, CP))
    dinv_ref[...] = dinv
    g_ref[...] = dinv * h_ref[...]
    z_ref[...] = ALPHA * dinv * h_ref[...]


def _final_body(a0_ref, a1_ref, g_ref, dinv_ref, x0_ref, o_ref):
    s = a0_ref[...] + a1_ref[...] + g_ref[...]
    h = (1.0 - ALPHA) * dinv_ref[...] * s + ALPHA * x0_ref[...]
    m = jnp.max(h, axis=1, keepdims=True)
    e = jnp.exp(h - m)
    o_ref[...] = h - m - jnp.log(jnp.sum(e, axis=1, keepdims=True))


def _mlp(x, W1, b1, W2, b2):
    return pl.pallas_call(
        _mlp_body,
        out_shape=jax.ShapeDtypeStruct((N, C), jnp.float32),
    )(x, W1, b1, W2, b2)


def _prep(a0, a1, h):
    return pl.pallas_call(
        _prep_body,
        out_shape=[
            jax.ShapeDtypeStruct((NP, CP), jnp.float32),
            jax.ShapeDtypeStruct((NP, CP), jnp.float32),
            jax.ShapeDtypeStruct((NP, CP), jnp.float32),
            jax.ShapeDtypeStruct((NP, 1), jnp.float32),
        ],
    )(a0, a1, h)


def _final(a0, a1, g, dinv, x0):
    return pl.pallas_call(
        _final_body,
        out_shape=jax.ShapeDtypeStruct((N, C), jnp.float32),
    )(a0, a1, g, dinv, x0)


_SC_MESH = plsc.VectorSubcoreMesh(core_axis_name="c", subcore_axis_name="s")

_EDGE_SCRATCH = [
    pltpu.VMEM((NCH, CHUNK), jnp.int32),
    pltpu.VMEM((NCH, CHUNK), jnp.int32),
    pltpu.VMEM((NBUF, CHUNK, CP), jnp.float32),
    pltpu.VMEM_SHARED((NP, CP), jnp.float32),
    pltpu.VMEM_SHARED((NP, CP), jnp.float32),
    pltpu.SemaphoreType.DMA((NBUF,)),
    pltpu.SemaphoreType.DMA((NBUF,)),
]


def _edge_phase(idx_r, idx_c, buf, acc, gtab, gsem, ssem):
    def gather_start(j, b):
        pltpu.async_copy(gtab.at[idx_r.at[j]], buf.at[b], gsem.at[b])

    def gather_wait(j, b):
        pltpu.make_async_copy(gtab.at[idx_r.at[j]], buf.at[b], gsem.at[b]).wait()

    def scatter_start(j, b):
        pltpu.async_copy(buf.at[b], acc.at[idx_c.at[j]], ssem.at[b], add=True)

    def scatter_wait(j, b):
        pltpu.make_async_copy(buf.at[b], acc.at[idx_c.at[j]], ssem.at[b]).wait()

    for b in range(NBUF):
        gather_start(b, b)

    def body(g, carry):
        jbase = g * NBUF
        for b in range(NBUF):
            gather_wait(jbase + b, b)
            scatter_start(jbase + b, b)
        for b in range(NBUF):
            scatter_wait(jbase + b, b)
            gather_start(jbase + NBUF + b, b)
        return carry

    lax.fori_loop(0, NG - 1, body, 0)
    jbase = (NG - 1) * NBUF
    for b in range(NBUF):
        gather_wait(jbase + b, b)
        scatter_start(jbase + b, b)
    for b in range(NBUF):
        scatter_wait(jbase + b, b)


@functools.partial(
    pl.kernel,
    out_type=jax.ShapeDtypeStruct((NCORE, NP, CP), jnp.float32),
    mesh=_SC_MESH,
    compiler_params=pltpu.CompilerParams(use_tc_tiling_on_sc=False),
    scratch_types=_EDGE_SCRATCH,
)
def _edge_pass(g_hbm, row_hbm, col_hbm, zeros_hbm, out_hbm,
               idx_r, idx_c, buf, acc, gtab, gsem, ssem):
    c = lax.axis_index("c")
    s = lax.axis_index("s")
    wid = c * NSUB + s
    base = pl.multiple_of(s * RPS, 8)
    pltpu.sync_copy(zeros_hbm.at[pl.ds(base, RPS)], acc.at[pl.ds(base, RPS)])
    pltpu.sync_copy(g_hbm.at[pl.ds(base, RPS)], gtab.at[pl.ds(base, RPS)])
    pltpu.sync_copy(row_hbm.at[wid], idx_r)
    pltpu.sync_copy(col_hbm.at[wid], idx_c)
    plsc.subcore_barrier()
    _edge_phase(idx_r, idx_c, buf, acc, gtab, gsem, ssem)
    plsc.subcore_barrier()
    pltpu.sync_copy(acc.at[pl.ds(base, RPS)], out_hbm.at[c, pl.ds(base, RPS)])


@functools.partial(
    pl.kernel,
    out_type=[
        jax.ShapeDtypeStruct((NCORE, NP, CP), jnp.float32),
        jax.ShapeDtypeStruct((NP, CP), jnp.float32),
    ],
    mesh=_SC_MESH,
    compiler_params=pltpu.CompilerParams(use_tc_tiling_on_sc=False),
    scratch_types=_EDGE_SCRATCH + [
        pltpu.VMEM((QR, CP), jnp.float32),
        pltpu.VMEM((QR, CP), jnp.float32),
        pltpu.VMEM((QR, CP), jnp.float32),
        pltpu.VMEM((QR, CP), jnp.float32),
        pltpu.VMEM((QR, CP), jnp.float32),
        pltpu.VMEM((QR, CP), jnp.float32),
        pltpu.SemaphoreType.DMA,
    ],
)
def _edge_pass_fused(a0_hbm, a1_hbm, gp_hbm, u_hbm, z_hbm, row_hbm, col_hbm,
                     zeros_hbm, out_hbm, gnew_hbm,
                     idx_r, idx_c, buf, acc, gtab, gsem, ssem,
                     va0, va1, vg, vu, vz, vo, psem):
    c = lax.axis_index("c")
    s = lax.axis_index("s")
    wid = c * NSUB + s
    base = pl.multiple_of(s * RPS, 8)
    pltpu.sync_copy(zeros_hbm.at[pl.ds(base, RPS)], acc.at[pl.ds(base, RPS)])
    pltpu.sync_copy(row_hbm.at[wid], idx_r)
    pltpu.sync_copy(col_hbm.at[wid], idx_c)

    for q in range(8):
        r0 = base + q * QR
        pltpu.async_copy(a0_hbm.at[pl.ds(r0, QR)], va0, psem)
        pltpu.async_copy(a1_hbm.at[pl.ds(r0, QR)], va1, psem)
        pltpu.async_copy(gp_hbm.at[pl.ds(r0, QR)], vg, psem)
        pltpu.async_copy(u_hbm.at[pl.ds(r0, QR)], vu, psem)
        pltpu.async_copy(z_hbm.at[pl.ds(r0, QR)], vz, psem)
        for dst in (va0, va1, vg, vu, vz):
            pltpu.make_async_copy(a0_hbm.at[pl.ds(r0, QR)], dst, psem).wait()

        def rowbody(r, carry):
            for kk in range(CP // 16):
                sl = pl.ds(kk * 16, 16)
                vo[r, sl] = vu[r, sl] * (va0[r, sl] + va1[r, sl] + vg[r, sl]) + vz[r, sl]
            return carry

        lax.fori_loop(0, QR, rowbody, 0)
        pltpu.sync_copy(vo, gtab.at[pl.ds(r0, QR)])

        @pl.when(c == 0)
        def _():
            pltpu.sync_copy(vo, gnew_hbm.at[pl.ds(r0, QR)])

    plsc.subcore_barrier()
    _edge_phase(idx_r, idx_c, buf, acc, gtab, gsem, ssem)
    plsc.subcore_barrier()
    pltpu.sync_copy(acc.at[pl.ds(base, RPS)], out_hbm.at[c, pl.ds(base, RPS)])


def kernel(x, edge_index, W1, b1, W2, b2):
    h0 = _mlp(x, W1, b1.reshape(1, H), W2, b2.reshape(1, C))
    h = jnp.pad(h0, ((0, NP - N), (0, CP - C)))

    pad_e = EP - E
    rp = jnp.concatenate(
        [edge_index[0], jnp.zeros((pad_e,), jnp.int32)]).reshape(NW, NCH, CHUNK)
    cp = jnp.concatenate(
        [edge_index[1], jnp.full((pad_e,), DUMMY, jnp.int32)]).reshape(NW, NCH, CHUNK)
    zeros = jnp.zeros((NP, CP), jnp.float32)
    ones = jnp.ones((NP, CP), jnp.float32)

    accd = _edge_pass(ones, rp, cp, zeros)
    g, z, u, dinv = _prep(accd[0], accd[1], h)
    acc = _edge_pass(g, rp, cp, zeros)
    for _ in range(K - 1):
        acc, g = _edge_pass_fused(acc[0], acc[1], g, u, z, rp, cp, zeros)
    return _final(acc[0, :N, :C], acc[1, :N, :C], g[:N, :C], dinv[:N], h0)


# speedup vs baseline: 1.2490x; 1.2490x over previous
import functools

import jax
import jax.numpy as jnp
from jax import lax
from jax.experimental import pallas as pl
from jax.experimental.pallas import tpu as pltpu
from jax.experimental.pallas import tpu_sc as plsc

N = 10000
D = 128
H = 64
C = 40
K = 10
ALPHA = 0.1
E = 320000
CP = 40

NSUB = 16
NCORE = 2
NP = 10112
RPS = NP // NSUB
NW = NCORE * NSUB
CHUNK = 128
NCH = 80
EP = NW * NCH * CHUNK
DUMMY = N
NBUF = 8
NG = NCH // NBUF


def _mlp_body(x_ref, w1_ref, b1_ref, w2_ref, b2_ref, o_ref):
    h = jnp.dot(x_ref[...], w1_ref[...], preferred_element_type=jnp.float32)
    h = jnp.maximum(h + b1_ref[...], 0.0)
    o_ref[...] = jnp.dot(h, w2_ref[...], preferred_element_type=jnp.float32) + b2_ref[...]


def _prep_body(a0_ref, a1_ref, h_ref, g_ref, z_ref, u_ref, dinv_ref):
    deg = a0_ref[:, 0:1] + a1_ref[:, 0:1] + 1.0
    dinv = lax.rsqrt(deg)
    u_ref[...] = 1.0 / deg
    dinv_ref[...] = dinv
    g_ref[...] = dinv * h_ref[...]
    z_ref[...] = ALPHA * dinv * h_ref[...]


def _update_body(a0_ref, a1_ref, g_ref, u_ref, z_ref, o_ref):
    s = a0_ref[...] + a1_ref[...] + g_ref[...]
    o_ref[...] = (1.0 - ALPHA) * u_ref[...] * s + z_ref[...]


def _final_body(a0_ref, a1_ref, g_ref, dinv_ref, x0_ref, o_ref):
    s = a0_ref[...] + a1_ref[...] + g_ref[...]
    h = (1.0 - ALPHA) * dinv_ref[...] * s + ALPHA * x0_ref[...]
    m = jnp.max(h, axis=1, keepdims=True)
    e = jnp.exp(h - m)
    o_ref[...] = h - m - jnp.log(jnp.sum(e, axis=1, keepdims=True))


def _mlp(x, W1, b1, W2, b2):
    return pl.pallas_call(
        _mlp_body,
        out_shape=jax.ShapeDtypeStruct((N, C), jnp.float32),
    )(x, W1, b1, W2, b2)


def _prep(a0, a1, h):
    return pl.pallas_call(
        _prep_body,
        out_shape=[
            jax.ShapeDtypeStruct((NP, CP), jnp.float32),
            jax.ShapeDtypeStruct((NP, CP), jnp.float32),
            jax.ShapeDtypeStruct((NP, 1), jnp.float32),
            jax.ShapeDtypeStruct((NP, 1), jnp.float32),
        ],
    )(a0, a1, h)


def _update(a0, a1, g, u, z):
    return pl.pallas_call(
        _update_body,
        out_shape=jax.ShapeDtypeStruct((NP, CP), jnp.float32),
    )(a0, a1, g, u, z)


def _final(a0, a1, g, dinv, x0):
    return pl.pallas_call(
        _final_body,
        out_shape=jax.ShapeDtypeStruct((N, C), jnp.float32),
    )(a0, a1, g, dinv, x0)


@functools.partial(
    pl.kernel,
    out_type=jax.ShapeDtypeStruct((NCORE, NP, CP), jnp.float32),
    mesh=plsc.VectorSubcoreMesh(core_axis_name="c", subcore_axis_name="s"),
    compiler_params=pltpu.CompilerParams(use_tc_tiling_on_sc=False),
    scratch_types=[
        pltpu.VMEM((NCH, CHUNK), jnp.int32),
        pltpu.VMEM((NCH, CHUNK), jnp.int32),
        pltpu.VMEM((NBUF, CHUNK, CP), jnp.float32),
        pltpu.VMEM_SHARED((NP, CP), jnp.float32),
        pltpu.VMEM_SHARED((NP, CP), jnp.float32),
        pltpu.SemaphoreType.DMA((NBUF,)),
        pltpu.SemaphoreType.DMA((NBUF,)),
        pltpu.SemaphoreType.DMA,
    ],
)
def _edge_pass(g_hbm, row_hbm, col_hbm, zeros_hbm, out_hbm,
               idx_r, idx_c, buf, acc, gtab, gsem, ssem, psem):
    c = lax.axis_index("c")
    s = lax.axis_index("s")
    wid = c * NSUB + s
    base = pl.multiple_of(s * RPS, 8)
    pltpu.async_copy(zeros_hbm.at[pl.ds(base, RPS)], acc.at[pl.ds(base, RPS)], psem)
    pltpu.async_copy(g_hbm.at[pl.ds(base, RPS)], gtab.at[pl.ds(base, RPS)], psem)
    pltpu.async_copy(row_hbm.at[wid], idx_r, psem)
    pltpu.async_copy(col_hbm.at[wid], idx_c, psem)
    pltpu.make_async_copy(zeros_hbm.at[pl.ds(base, RPS)], acc.at[pl.ds(base, RPS)], psem).wait()
    pltpu.make_async_copy(g_hbm.at[pl.ds(base, RPS)], gtab.at[pl.ds(base, RPS)], psem).wait()
    pltpu.make_async_copy(row_hbm.at[wid], idx_r, psem).wait()
    pltpu.make_async_copy(col_hbm.at[wid], idx_c, psem).wait()
    plsc.subcore_barrier()

    def gather_start(j, b):
        pltpu.async_copy(gtab.at[idx_r.at[j]], buf.at[b], gsem.at[b])

    def gather_wait(j, b):
        pltpu.make_async_copy(gtab.at[idx_r.at[j]], buf.at[b], gsem.at[b]).wait()

    def scatter_start(j, b):
        pltpu.async_copy(buf.at[b], acc.at[idx_c.at[j]], ssem.at[b], add=True)

    def scatter_wait(j, b):
        pltpu.make_async_copy(buf.at[b], acc.at[idx_c.at[j]], ssem.at[b]).wait()

    for b in range(NBUF):
        gather_start(b, b)

    def body(g, carry):
        jbase = g * NBUF
        for b in range(NBUF):
            gather_wait(jbase + b, b)
            scatter_start(jbase + b, b)
        for b in range(NBUF):
            scatter_wait(jbase + b, b)
            gather_start(jbase + NBUF + b, b)
        return carry

    lax.fori_loop(0, NG - 1, body, 0)
    jbase = (NG - 1) * NBUF
    for b in range(NBUF):
        gather_wait(jbase + b, b)
        scatter_start(jbase + b, b)
    for b in range(NBUF):
        scatter_wait(jbase + b, b)
    plsc.subcore_barrier()
    pltpu.sync_copy(acc.at[pl.ds(base, RPS)], out_hbm.at[c, pl.ds(base, RPS)])


def kernel(x, edge_index, W1, b1, W2, b2):
    h0 = _mlp(x, W1, b1.reshape(1, H), W2, b2.reshape(1, C))
    h = jnp.pad(h0, ((0, NP - N), (0, CP - C)))

    pad_e = EP - E
    rp = jnp.concatenate(
        [edge_index[0], jnp.zeros((pad_e,), jnp.int32)]).reshape(NW, NCH, CHUNK)
    cp = jnp.concatenate(
        [edge_index[1], jnp.full((pad_e,), DUMMY, jnp.int32)]).reshape(NW, NCH, CHUNK)
    zeros = jnp.zeros((NP, CP), jnp.float32)
    ones = jnp.ones((NP, CP), jnp.float32)

    accd = _edge_pass(ones, rp, cp, zeros)
    g, z, u, dinv = _prep(accd[0], accd[1], h)
    for _ in range(K - 1):
        acc = _edge_pass(g, rp, cp, zeros)
        g = _update(acc[0], acc[1], g, u, z)
    acc = _edge_pass(g, rp, cp, zeros)
    return _final(acc[0, :N, :C], acc[1, :N, :C], g[:N, :C], dinv[:N], h0)


# speedup vs baseline: 1.3028x; 1.0431x over previous
import functools

import jax
import jax.numpy as jnp
from jax import lax
from jax.experimental import pallas as pl
from jax.experimental.pallas import tpu as pltpu
from jax.experimental.pallas import tpu_sc as plsc

N = 10000
D = 128
H = 64
C = 40
K = 10
ALPHA = 0.1
E = 320000
CP = 40

NSUB = 16
NCORE = 2
NP = 10112
RPS = NP // NSUB
NW = NCORE * NSUB
CHUNK = 128
NCH = 80
EP = NW * NCH * CHUNK
DUMMY = N
NBUF = 10
NG = NCH // NBUF


def _mlp_body(x_ref, w1_ref, b1_ref, w2_ref, b2_ref, o_ref):
    h = jnp.dot(x_ref[...], w1_ref[...], preferred_element_type=jnp.float32)
    h = jnp.maximum(h + b1_ref[...], 0.0)
    o_ref[...] = jnp.dot(h, w2_ref[...], preferred_element_type=jnp.float32) + b2_ref[...]


def _prep_body(a0_ref, a1_ref, h_ref, g_ref, z_ref, u_ref, dinv_ref):
    deg = a0_ref[:, 0:1] + a1_ref[:, 0:1] + 1.0
    dinv = lax.rsqrt(deg)
    u_ref[...] = 1.0 / deg
    dinv_ref[...] = dinv
    g_ref[...] = dinv * h_ref[...]
    z_ref[...] = ALPHA * dinv * h_ref[...]


def _update_body(a0_ref, a1_ref, g_ref, u_ref, z_ref, o_ref):
    s = a0_ref[...] + a1_ref[...] + g_ref[...]
    o_ref[...] = (1.0 - ALPHA) * u_ref[...] * s + z_ref[...]


def _final_body(a0_ref, a1_ref, g_ref, dinv_ref, x0_ref, o_ref):
    s = a0_ref[...] + a1_ref[...] + g_ref[...]
    h = (1.0 - ALPHA) * dinv_ref[...] * s + ALPHA * x0_ref[...]
    m = jnp.max(h, axis=1, keepdims=True)
    e = jnp.exp(h - m)
    o_ref[...] = h - m - jnp.log(jnp.sum(e, axis=1, keepdims=True))


def _mlp(x, W1, b1, W2, b2):
    return pl.pallas_call(
        _mlp_body,
        out_shape=jax.ShapeDtypeStruct((N, C), jnp.float32),
    )(x, W1, b1, W2, b2)


def _prep(a0, a1, h):
    return pl.pallas_call(
        _prep_body,
        out_shape=[
            jax.ShapeDtypeStruct((NP, CP), jnp.float32),
            jax.ShapeDtypeStruct((NP, CP), jnp.float32),
            jax.ShapeDtypeStruct((NP, 1), jnp.float32),
            jax.ShapeDtypeStruct((NP, 1), jnp.float32),
        ],
    )(a0, a1, h)


def _update(a0, a1, g, u, z):
    return pl.pallas_call(
        _update_body,
        out_shape=jax.ShapeDtypeStruct((NP, CP), jnp.float32),
    )(a0, a1, g, u, z)


def _final(a0, a1, g, dinv, x0):
    return pl.pallas_call(
        _final_body,
        out_shape=jax.ShapeDtypeStruct((N, C), jnp.float32),
    )(a0, a1, g, dinv, x0)


@functools.partial(
    pl.kernel,
    out_type=jax.ShapeDtypeStruct((NCORE, NP, CP), jnp.float32),
    mesh=plsc.VectorSubcoreMesh(core_axis_name="c", subcore_axis_name="s"),
    compiler_params=pltpu.CompilerParams(use_tc_tiling_on_sc=False),
    scratch_types=[
        pltpu.VMEM((NCH, CHUNK), jnp.int32),
        pltpu.VMEM((CHUNK, CP), jnp.float32),
        pltpu.VMEM_SHARED((NP, CP), jnp.float32),
        pltpu.SemaphoreType.DMA,
        pltpu.SemaphoreType.DMA,
    ],
)
def _deg_pass(ones_hbm, col_hbm, zeros_hbm, out_hbm, idx_c, buf, acc, ssem, psem):
    c = lax.axis_index("c")
    s = lax.axis_index("s")
    wid = c * NSUB + s
    base = pl.multiple_of(s * RPS, 8)
    pltpu.async_copy(zeros_hbm.at[pl.ds(base, RPS)], acc.at[pl.ds(base, RPS)], psem)
    pltpu.async_copy(col_hbm.at[wid], idx_c, psem)
    pltpu.async_copy(ones_hbm, buf, psem)
    pltpu.make_async_copy(zeros_hbm.at[pl.ds(base, RPS)], acc.at[pl.ds(base, RPS)], psem).wait()
    pltpu.make_async_copy(col_hbm.at[wid], idx_c, psem).wait()
    pltpu.make_async_copy(ones_hbm, buf, psem).wait()
    plsc.subcore_barrier()

    def fire(j, carry):
        pltpu.async_copy(buf, acc.at[idx_c.at[j]], ssem, add=True)
        return carry

    lax.fori_loop(0, NCH, fire, 0)

    def drain(j, carry):
        pltpu.make_async_copy(buf, acc.at[idx_c.at[j]], ssem).wait()
        return carry

    lax.fori_loop(0, NCH, drain, 0)
    plsc.subcore_barrier()
    pltpu.sync_copy(acc.at[pl.ds(base, RPS)], out_hbm.at[c, pl.ds(base, RPS)])


@functools.partial(
    pl.kernel,
    out_type=jax.ShapeDtypeStruct((NCORE, NP, CP), jnp.float32),
    mesh=plsc.VectorSubcoreMesh(core_axis_name="c", subcore_axis_name="s"),
    compiler_params=pltpu.CompilerParams(use_tc_tiling_on_sc=False),
    scratch_types=[
        pltpu.VMEM((NCH, CHUNK), jnp.int32),
        pltpu.VMEM((NCH, CHUNK), jnp.int32),
        pltpu.VMEM((NBUF, CHUNK, CP), jnp.float32),
        pltpu.VMEM_SHARED((NP, CP), jnp.float32),
        pltpu.VMEM_SHARED((NP, CP), jnp.float32),
        pltpu.SemaphoreType.DMA((NBUF,)),
        pltpu.SemaphoreType.DMA((NBUF,)),
        pltpu.SemaphoreType.DMA,
    ],
)
def _edge_pass(g_hbm, row_hbm, col_hbm, zeros_hbm, out_hbm,
               idx_r, idx_c, buf, acc, gtab, gsem, ssem, psem):
    c = lax.axis_index("c")
    s = lax.axis_index("s")
    wid = c * NSUB + s
    base = pl.multiple_of(s * RPS, 8)
    pltpu.async_copy(zeros_hbm.at[pl.ds(base, RPS)], acc.at[pl.ds(base, RPS)], psem)
    pltpu.async_copy(g_hbm.at[pl.ds(base, RPS)], gtab.at[pl.ds(base, RPS)], psem)
    pltpu.async_copy(row_hbm.at[wid], idx_r, psem)
    pltpu.async_copy(col_hbm.at[wid], idx_c, psem)
    pltpu.make_async_copy(zeros_hbm.at[pl.ds(base, RPS)], acc.at[pl.ds(base, RPS)], psem).wait()
    pltpu.make_async_copy(g_hbm.at[pl.ds(base, RPS)], gtab.at[pl.ds(base, RPS)], psem).wait()
    pltpu.make_async_copy(row_hbm.at[wid], idx_r, psem).wait()
    pltpu.make_async_copy(col_hbm.at[wid], idx_c, psem).wait()
    plsc.subcore_barrier()

    def gather_start(j, b):
        pltpu.async_copy(gtab.at[idx_r.at[j]], buf.at[b], gsem.at[b])

    def gather_wait(j, b):
        pltpu.make_async_copy(gtab.at[idx_r.at[j]], buf.at[b], gsem.at[b]).wait()

    def scatter_start(j, b):
        pltpu.async_copy(buf.at[b], acc.at[idx_c.at[j]], ssem.at[b], add=True)

    def scatter_wait(j, b):
        pltpu.make_async_copy(buf.at[b], acc.at[idx_c.at[j]], ssem.at[b]).wait()

    for b in range(NBUF):
        gather_start(b, b)

    def body(g, carry):
        jbase = g * NBUF
        for b in range(NBUF):
            gather_wait(jbase + b, b)
            scatter_start(jbase + b, b)
        for b in range(NBUF):
            scatter_wait(jbase + b, b)
            gather_start(jbase + NBUF + b, b)
        return carry

    lax.fori_loop(0, NG - 1, body, 0)
    jbase = (NG - 1) * NBUF
    for b in range(NBUF):
        gather_wait(jbase + b, b)
        scatter_start(jbase + b, b)
    for b in range(NBUF):
        scatter_wait(jbase + b, b)
    plsc.subcore_barrier()
    pltpu.sync_copy(acc.at[pl.ds(base, RPS)], out_hbm.at[c, pl.ds(base, RPS)])


def kernel(x, edge_index, W1, b1, W2, b2):
    h0 = _mlp(x, W1, b1.reshape(1, H), W2, b2.reshape(1, C))
    h = jnp.pad(h0, ((0, NP - N), (0, CP - C)))

    pad_e = EP - E
    rp = jnp.concatenate(
        [edge_index[0], jnp.zeros((pad_e,), jnp.int32)]).reshape(NW, NCH, CHUNK)
    cp = jnp.concatenate(
        [edge_index[1], jnp.full((pad_e,), DUMMY, jnp.int32)]).reshape(NW, NCH, CHUNK)
    zeros = jnp.zeros((NP, CP), jnp.float32)
    ones = jnp.ones((CHUNK, CP), jnp.float32)

    accd = _deg_pass(ones, cp, zeros)
    g, z, u, dinv = _prep(accd[0], accd[1], h)
    for _ in range(K - 1):
        acc = _edge_pass(g, rp, cp, zeros)
        g = _update(acc[0], acc[1], g, u, z)
    acc = _edge_pass(g, rp, cp, zeros)
    return _final(acc[0, :N, :C], acc[1, :N, :C], g[:N, :C], dinv[:N], h0)
